# baseline (speedup 1.0000x reference)
import functools

import jax
import jax.numpy as jnp
from jax import lax
from jax.experimental import pallas as pl
from jax.experimental.pallas import tpu as pltpu
from jax.experimental.pallas import tpu_sc as plsc

N_NODES = 100000
ATOM_DIM = 32
CHARGE_DIM = 16
OUT_DIM = ATOM_DIM + CHARGE_DIM
LANE = 128

NUM_CORES = 2
NUM_SUBCORES = 16
NW = NUM_CORES * NUM_SUBCORES

BPW = 3200
LAST_BASE = N_NODES - BPW
CHUNK = 400
NCHUNK = BPW // CHUNK

_mesh = plsc.VectorSubcoreMesh(core_axis_name="c", subcore_axis_name="s")


@functools.partial(
    pl.kernel,
    mesh=_mesh,
    compiler_params=pltpu.CompilerParams(use_tc_tiling_on_sc=False),
    out_type=jax.ShapeDtypeStruct((N_NODES * LANE,), jnp.float32),
    scratch_types=[
        pltpu.VMEM((BPW,), jnp.int32),
        pltpu.VMEM((BPW,), jnp.int32),
        pltpu.VMEM((CHUNK, ATOM_DIM), jnp.float32),
        pltpu.VMEM((CHUNK, CHARGE_DIM), jnp.float32),
        pltpu.VMEM((CHUNK * LANE,), jnp.float32),
        pltpu.SemaphoreType.DMA,
        pltpu.SemaphoreType.DMA,
    ],
)
def _sc_embed(aidx_hbm, cidx_hbm, atable_hbm, ctable_hbm, out_hbm,
              aidx_v, cidx_v, arows_v, crows_v, comb_v, sem_a, sem_c):
    wid = lax.axis_index("s") * NUM_CORES + lax.axis_index("c")
    base = jnp.minimum(wid * BPW, LAST_BASE)
    pltpu.sync_copy(aidx_hbm.at[pl.ds(base, BPW)], aidx_v)
    pltpu.sync_copy(cidx_hbm.at[pl.ds(base, BPW)], cidx_v)

    def body(j, carry):
        off = j * CHUNK
        cp_a = pltpu.async_copy(
            atable_hbm.at[aidx_v.at[pl.ds(off, CHUNK)]], arows_v, sem_a)
        cp_c = pltpu.async_copy(
            ctable_hbm.at[cidx_v.at[pl.ds(off, CHUNK)]], crows_v, sem_c)
        cp_a.wait()
        cp_c.wait()

        def merge(r, c2):
            comb_v[pl.ds(r * LANE, 16)] = arows_v[r, pl.ds(0, 16)]
            comb_v[pl.ds(r * LANE + 16, 16)] = arows_v[r, pl.ds(16, 16)]
            comb_v[pl.ds(r * LANE + 32, 16)] = crows_v[r, pl.ds(0, 16)]
            return c2

        lax.fori_loop(0, CHUNK, merge, 0)
        pltpu.sync_copy(
            comb_v, out_hbm.at[pl.ds((base + off) * LANE, CHUNK * LANE)])
        return carry

    lax.fori_loop(0, NCHUNK, body, 0)


TC_ROWS = 2000


def _tc_relayout_body(x_ref, o_ref):
    o_ref[...] = x_ref[...].reshape(TC_ROWS, LANE)[:, :OUT_DIM]


_tc_relayout = pl.pallas_call(
    _tc_relayout_body,
    grid=(N_NODES // TC_ROWS,),
    in_specs=[pl.BlockSpec((TC_ROWS * LANE,), lambda i: (i,))],
    out_specs=pl.BlockSpec((TC_ROWS, OUT_DIM), lambda i: (i, 0)),
    out_shape=jax.ShapeDtypeStruct((N_NODES, OUT_DIM), jnp.float32),
)


def kernel(atom_types, charge, atom_types_table, charge_table):
    aidx = atom_types.reshape(-1).astype(jnp.int32)
    cidx = charge.reshape(-1).astype(jnp.int32)
    out1 = _sc_embed(aidx, cidx, atom_types_table, charge_table)
    return _tc_relayout(out1)


# speedup vs baseline: 1.0476x; 1.0476x over previous
import functools

import jax
import jax.numpy as jnp
from jax import lax
from jax.experimental import pallas as pl
from jax.experimental.pallas import tpu as pltpu
from jax.experimental.pallas import tpu_sc as plsc

N_NODES = 100000
ATOM_DIM = 32
CHARGE_DIM = 16
CHARGE_VOCAB = 100
OUT_DIM = ATOM_DIM + CHARGE_DIM
LANE = 128

NUM_CORES = 2
NUM_SUBCORES = 16
NW = NUM_CORES * NUM_SUBCORES

BPW = 3200
LAST_BASE = N_NODES - BPW
CHUNK = 400
NCHUNK = BPW // CHUNK

_mesh = plsc.VectorSubcoreMesh(core_axis_name="c", subcore_axis_name="s")


@functools.partial(
    pl.kernel,
    mesh=_mesh,
    compiler_params=pltpu.CompilerParams(use_tc_tiling_on_sc=False),
    out_type=jax.ShapeDtypeStruct((N_NODES * LANE,), jnp.float32),
    scratch_types=[
        pltpu.VMEM((BPW,), jnp.int32),
        pltpu.VMEM((CHUNK, ATOM_DIM), jnp.float32),
        pltpu.VMEM((CHUNK * LANE,), jnp.float32),
        pltpu.SemaphoreType.DMA,
    ],
)
def _sc_gather(aidx_hbm, atable_hbm, out_hbm, aidx_v, arows_v, comb_v, sem):
    wid = lax.axis_index("s") * NUM_CORES + lax.axis_index("c")
    base = jnp.minimum(wid * BPW, LAST_BASE)
    pltpu.sync_copy(aidx_hbm.at[pl.ds(base, BPW)], aidx_v)

    def body(j, carry):
        off = j * CHUNK
        pltpu.async_copy(
            atable_hbm.at[aidx_v.at[pl.ds(off, CHUNK)]], arows_v, sem).wait()

        def merge(r, c2):
            comb_v[pl.ds(r * LANE, 16)] = arows_v[r, pl.ds(0, 16)]
            comb_v[pl.ds(r * LANE + 16, 16)] = arows_v[r, pl.ds(16, 16)]
            return c2

        lax.fori_loop(0, CHUNK, merge, 0)
        pltpu.sync_copy(
            comb_v, out_hbm.at[pl.ds((base + off) * LANE, CHUNK * LANE)])
        return carry

    lax.fori_loop(0, NCHUNK, body, 0)


TC_ROWS = 2048


def _tc_body(x_ref, cidx_ref, ctab_ref, o_ref):
    x = x_ref[...].reshape(TC_ROWS, LANE)
    cid = cidx_ref[...].reshape(TC_ROWS, 1)
    hot = (cid == lax.broadcasted_iota(jnp.int32, (1, CHARGE_VOCAB), 1))
    cvals = jnp.dot(hot.astype(jnp.float32), ctab_ref[...],
                    preferred_element_type=jnp.float32)
    o_ref[...] = jnp.concatenate([x[:, :ATOM_DIM], cvals], axis=1)


_tc_finish = pl.pallas_call(
    _tc_body,
    grid=((N_NODES + TC_ROWS - 1) // TC_ROWS,),
    in_specs=[
        pl.BlockSpec((TC_ROWS * LANE,), lambda i: (i,)),
        pl.BlockSpec((TC_ROWS,), lambda i: (i,)),
        pl.BlockSpec((CHARGE_VOCAB, CHARGE_DIM), lambda i: (0, 0)),
    ],
    out_specs=pl.BlockSpec((TC_ROWS, OUT_DIM), lambda i: (i, 0)),
    out_shape=jax.ShapeDtypeStruct((N_NODES, OUT_DIM), jnp.float32),
)


def kernel(atom_types, charge, atom_types_table, charge_table):
    aidx = atom_types.reshape(-1).astype(jnp.int32)
    cidx = charge.reshape(-1).astype(jnp.int32)
    apad = _sc_gather(aidx, atom_types_table)
    return _tc_finish(apad, cidx, charge_table)


# speedup vs baseline: 1.0913x; 1.0417x over previous
import functools

import jax
import jax.numpy as jnp
from jax import lax
from jax.experimental import pallas as pl
from jax.experimental.pallas import tpu as pltpu
from jax.experimental.pallas import tpu_sc as plsc

N_NODES = 100000
ATOM_DIM = 32
CHARGE_DIM = 16
OUT_DIM = ATOM_DIM + CHARGE_DIM
LANE = 128

NUM_CORES = 2
NUM_SUBCORES = 16
NW = NUM_CORES * NUM_SUBCORES

BPW = 3200
LAST_BASE = N_NODES - BPW
CHUNK = 400
NCHUNK = BPW // CHUNK

_mesh = plsc.VectorSubcoreMesh(core_axis_name="c", subcore_axis_name="s")


@functools.partial(
    pl.kernel,
    mesh=_mesh,
    compiler_params=pltpu.CompilerParams(use_tc_tiling_on_sc=False),
    out_type=jax.ShapeDtypeStruct((N_NODES // 8, 8, LANE), jnp.float32),
    scratch_types=[
        pltpu.VMEM((BPW,), jnp.int32),
        pltpu.VMEM((BPW,), jnp.int32),
        pltpu.VMEM((CHUNK, ATOM_DIM), jnp.float32),
        pltpu.VMEM((CHUNK, CHARGE_DIM), jnp.float32),
        pltpu.VMEM((CHUNK // 8, 8, LANE), jnp.float32),
        pltpu.SemaphoreType.DMA,
        pltpu.SemaphoreType.DMA,
    ],
)
def _sc_embed(aidx_hbm, cidx_hbm, atable_hbm, ctable_hbm, out_hbm,
              aidx_v, cidx_v, arows_v, crows_v, comb_v, sem_a, sem_c):
    wid = lax.axis_index("s") * NUM_CORES + lax.axis_index("c")
    base = jnp.minimum(wid * BPW, LAST_BASE)
    pltpu.sync_copy(aidx_hbm.at[pl.ds(base, BPW)], aidx_v)
    pltpu.sync_copy(cidx_hbm.at[pl.ds(base, BPW)], cidx_v)

    def body(j, carry):
        off = j * CHUNK
        cp_a = pltpu.async_copy(
            atable_hbm.at[aidx_v.at[pl.ds(off, CHUNK)]], arows_v, sem_a)
        cp_c = pltpu.async_copy(
            ctable_hbm.at[cidx_v.at[pl.ds(off, CHUNK)]], crows_v, sem_c)
        cp_a.wait()
        cp_c.wait()

        def merge(g, c2):
            for s in range(8):
                r = g * 8 + s
                comb_v[g, s, pl.ds(0, 16)] = arows_v[r, pl.ds(0, 16)]
                comb_v[g, s, pl.ds(16, 16)] = arows_v[r, pl.ds(16, 16)]
                comb_v[g, s, pl.ds(32, 16)] = crows_v[r, pl.ds(0, 16)]
            return c2

        lax.fori_loop(0, CHUNK // 8, merge, 0)
        pltpu.sync_copy(comb_v, out_hbm.at[pl.ds((base + off) // 8, CHUNK // 8)])
        return carry

    lax.fori_loop(0, NCHUNK, body, 0)


def kernel(atom_types, charge, atom_types_table, charge_table):
    aidx = atom_types.reshape(-1).astype(jnp.int32)
    cidx = charge.reshape(-1).astype(jnp.int32)
    out3 = _sc_embed(aidx, cidx, atom_types_table, charge_table)
    return out3.reshape(N_NODES, LANE)[:, :OUT_DIM]
